# baseline (speedup 1.0000x reference)
import functools

import jax
import jax.numpy as jnp
from jax import lax
from jax.experimental import pallas as pl
from jax.experimental.pallas import tpu as pltpu
from jax.experimental.pallas import tpu_sc as plsc

N = 10000
E = 160000
NB_TILE = 80
BB = 128
EP = 16 * NB_TILE * BB
NACC = 10112
HB = NB_TILE // 2
ZSPLIT = (128, 128, 128, 128, 120)

@functools.lru_cache(maxsize=None)
def _mesh():
  return plsc.VectorSubcoreMesh(
      core_axis_name="c", subcore_axis_name="s", num_cores=2, num_subcores=16)

BLK = 2000
GRID = N // BLK


def _zero_acc(acc, zb, s):
  off = 0
  for rows in ZSPLIT:
    pltpu.sync_copy(zb.at[pl.ds(0, rows)], acc.at[pl.ds(s * 632 + off, rows)])
    off += rows


def _sc_degree_body(dst_hbm, ones_hbm, z_hbm, deg_a, deg_b, acc, idxb, onesb,
                    zb):
  c = lax.axis_index("c")
  s = lax.axis_index("s")
  pltpu.sync_copy(dst_hbm.at[s, pl.ds(c * HB, HB)], idxb)
  pltpu.sync_copy(ones_hbm, onesb)
  pltpu.sync_copy(z_hbm, zb)
  _zero_acc(acc, zb, s)
  plsc.subcore_barrier()

  @pl.loop(0, HB)
  def _(b):
    pltpu.sync_copy(onesb, acc.at[idxb.at[b]], add=True)

  plsc.subcore_barrier()

  @pl.when(c == 0)
  def _():
    pltpu.sync_copy(acc.at[pl.ds(s * 632, 632)], deg_a.at[pl.ds(s * 632, 632)])

  @pl.when(c == 1)
  def _():
    pltpu.sync_copy(acc.at[pl.ds(s * 632, 632)], deg_b.at[pl.ds(s * 632, 632)])


def _sc_degree(dst3, ones128, zeros128):
  f = pl.kernel(
      _sc_degree_body,
      out_type=[jax.ShapeDtypeStruct((NACC, 128), jnp.float32)] * 2,
      mesh=_mesh(),
      scratch_types=[
          pltpu.VMEM_SHARED((NACC, 128), jnp.float32),
          pltpu.VMEM((HB, BB), jnp.int32),
          pltpu.VMEM((BB, 128), jnp.float32),
          pltpu.VMEM((BB, 128), jnp.float32),
      ],
  )
  return f(dst3, ones128, zeros128)


def _scatter_chunk(href, acc, src_hbm, dst_hbm, s, srcb, dstb, rowa, rowbuf2,
                   sema, semb, semsa, semsb):
  for half in range(2):
    pltpu.sync_copy(src_hbm.at[s, pl.ds(half * HB, HB)], srcb)
    pltpu.sync_copy(dst_hbm.at[s, pl.ds(half * HB, HB)], dstb)
    pltpu.async_copy(href.at[srcb.at[0]], rowa, sema)

    @pl.loop(0, HB, step=2, unroll=2)
    def _(b):
      pltpu.make_async_copy(href.at[srcb.at[b]], rowa, sema).wait()

      @pl.when(b > 0)
      def _():
        pltpu.make_async_copy(rowbuf2, acc.at[dstb.at[0]], semsb).wait()

      pltpu.async_copy(href.at[srcb.at[b + 1]], rowbuf2, semb)
      pltpu.async_copy(rowa, acc.at[dstb.at[b]], semsa, add=True)
      pltpu.make_async_copy(href.at[srcb.at[b + 1]], rowbuf2, semb).wait()
      pltpu.make_async_copy(rowa, acc.at[dstb.at[0]], semsa).wait()

      @pl.when(b + 2 < HB)
      def _():
        pltpu.async_copy(href.at[srcb.at[b + 2]], rowa, sema)

      pltpu.async_copy(rowbuf2, acc.at[dstb.at[b + 1]], semsb, add=True)

    pltpu.make_async_copy(rowbuf2, acc.at[dstb.at[0]], semsb).wait()


def _make_propagate_body(K):
  R = K // 2

  def body(*refs):
    hc = refs[:K]
    src_hbm, dst_hbm, z_hbm = refs[K:K + 3]
    outs = refs[K + 3:K + 3 + K]
    acc, srcb, dstb, rowa, rowbuf2, sema, semb, semsa, semsb = refs[K + 3 + K:]
    c = lax.axis_index("c")
    s = lax.axis_index("s")
    for r in range(R):
      pltpu.sync_copy(z_hbm, rowa)
      _zero_acc(acc, rowa, s)
      plsc.subcore_barrier()
      for k in range(K):
        @pl.when(c * R + r == k)
        def _(k=k):
          _scatter_chunk(hc[k], acc, src_hbm, dst_hbm, s, srcb, dstb, rowa,
                         rowbuf2, sema, semb, semsa, semsb)
      plsc.subcore_barrier()
      for k in range(K):
        @pl.when(c * R + r == k)
        def _(k=k):
          pltpu.sync_copy(acc.at[pl.ds(s * 632, 632)],
                          outs[k].at[pl.ds(s * 632, 632)])
      plsc.subcore_barrier()

  return body


def _sc_propagate(h_chunks, src3, dst3, zeros128):
  K = len(h_chunks)
  f = pl.kernel(
      _make_propagate_body(K),
      out_type=[jax.ShapeDtypeStruct((NACC, 128), jnp.float32)] * K,
      mesh=_mesh(),
      scratch_types=[
          pltpu.VMEM_SHARED((NACC, 128), jnp.float32),
          pltpu.VMEM((HB, BB), jnp.int32),
          pltpu.VMEM((HB, BB), jnp.int32),
          pltpu.VMEM((BB, 128), jnp.float32),
          pltpu.VMEM((BB, 128), jnp.float32),
          pltpu.SemaphoreType.DMA,
          pltpu.SemaphoreType.DMA,
          pltpu.SemaphoreType.DMA,
          pltpu.SemaphoreType.DMA,
      ],
  )
  return f(*h_chunks, src3, dst3, zeros128)


def _tc_pre(x, dega, degb):
  def body(x_ref, da_ref, db_ref, od, o0, o1):
    dinv = lax.rsqrt(da_ref[:, :1] + db_ref[:, :1] + 1.0)
    od[...] = jnp.broadcast_to(dinv, (BLK, 128))
    h = x_ref[...] * dinv
    o0[...] = h[:, :128]
    o1[...] = h[:, 128:]

  return pl.pallas_call(
      body,
      grid=(GRID,),
      in_specs=[pl.BlockSpec((BLK, 256), lambda i: (i, 0)),
                pl.BlockSpec((BLK, 128), lambda i: (i, 0)),
                pl.BlockSpec((BLK, 128), lambda i: (i, 0))],
      out_specs=[pl.BlockSpec((BLK, 128), lambda i: (i, 0))] * 3,
      out_shape=[jax.ShapeDtypeStruct((N, 128), jnp.float32)] * 3,
  )(x, dega, degb)


def _tc_layer(S_chunks, h_chunks, deg, W, b, g, be, W_extra=None):
  KI = len(S_chunks)
  Wout = W.shape[1]
  KO = (W_extra.shape[1] if W_extra is not None else Wout) // 128
  n_extra = 1 if W_extra is not None else 0

  def body(*refs):
    S = refs[:KI]
    h = refs[KI:2 * KI]
    deg_ref, W_ref, b_ref, g_ref, be_ref = refs[2 * KI:2 * KI + 5]
    We_ref = refs[2 * KI + 5] if n_extra else None
    outs = refs[2 * KI + 5 + n_extra:]
    dinv = deg_ref[:, :1]
    u = jnp.concatenate([S[i][...] + h[i][...] for i in range(KI)],
                        axis=1) * dinv
    z = jnp.dot(u, W_ref[...], preferred_element_type=jnp.float32) + b_ref[...]
    m = jnp.mean(z, axis=1, keepdims=True)
    d = z - m
    v = jnp.mean(d * d, axis=1, keepdims=True)
    a = d * lax.rsqrt(v + 1e-5) * g_ref[...] + be_ref[...]
    hn = jnp.maximum(a, 0.0) * dinv
    if n_extra:
      hn = jnp.dot(hn, We_ref[...], preferred_element_type=jnp.float32)
    for j in range(KO):
      outs[j][...] = hn[:, j * 128:(j + 1) * 128]

  Win = KI * 128
  in_specs = (
      [pl.BlockSpec((BLK, 128), lambda i: (i, 0))] * (2 * KI)
      + [pl.BlockSpec((BLK, 128), lambda i: (i, 0)),
         pl.BlockSpec((Win, Wout), lambda i: (0, 0)),
         pl.BlockSpec((1, Wout), lambda i: (0, 0)),
         pl.BlockSpec((1, Wout), lambda i: (0, 0)),
         pl.BlockSpec((1, Wout), lambda i: (0, 0))])
  args = list(S_chunks) + list(h_chunks) + [deg, W, b, g, be]
  if n_extra:
    in_specs.append(pl.BlockSpec(W_extra.shape, lambda i: (0, 0)))
    args.append(W_extra)
  return pl.pallas_call(
      body,
      grid=(GRID,),
      in_specs=in_specs,
      out_specs=[pl.BlockSpec((BLK, 128), lambda i: (i, 0))] * KO,
      out_shape=[jax.ShapeDtypeStruct((N, 128), jnp.float32)] * KO,
  )(*args)


def _tc_final(S_chunks, q_chunks, deg, b, g, be, batch):
  def body(S0, S1, q0, q1, deg_ref, b_ref, g_ref, be_ref, batch_ref,
           emb_out, ge_out, sums, cnts):
    i = pl.program_id(0)
    dinv = deg_ref[:, :1]
    u = jnp.concatenate(
        [S0[...] + q0[...], S1[...] + q1[...]], axis=1) * dinv + b_ref[...]
    m = jnp.mean(u, axis=1, keepdims=True)
    d = u - m
    v = jnp.mean(d * d, axis=1, keepdims=True)
    hn = d * lax.rsqrt(v + 1e-5) * g_ref[...] + be_ref[...]
    hn = jnp.where(hn > 0, hn, 0.2 * hn)
    emb_out[...] = hn

    bvec = batch_ref[...].reshape(BLK)
    oh = (lax.broadcasted_iota(jnp.int32, (16, BLK), 0)
          == bvec[None, :]).astype(jnp.float32)
    ps = jnp.dot(oh, hn, preferred_element_type=jnp.float32)
    pc = jnp.sum(oh, axis=1, keepdims=True)

    @pl.when(i == 0)
    def _():
      sums[...] = jnp.zeros_like(sums)
      cnts[...] = jnp.zeros_like(cnts)

    sums[...] += ps
    cnts[...] = cnts[...] + pc
    ge_out[...] = sums[...] / jnp.maximum(cnts[...], 1.0)

  return pl.pallas_call(
      body,
      grid=(GRID,),
      in_specs=([pl.BlockSpec((BLK, 128), lambda i: (i, 0))] * 4
                + [pl.BlockSpec((BLK, 128), lambda i: (i, 0)),
                   pl.BlockSpec((1, 256), lambda i: (0, 0)),
                   pl.BlockSpec((1, 256), lambda i: (0, 0)),
                   pl.BlockSpec((1, 256), lambda i: (0, 0)),
                   pl.BlockSpec((1, 1, BLK), lambda i: (i, 0, 0))]),
      out_specs=[pl.BlockSpec((BLK, 256), lambda i: (i, 0)),
                 pl.BlockSpec((16, 256), lambda i: (0, 0))],
      out_shape=[jax.ShapeDtypeStruct((N, 256), jnp.float32),
                 jax.ShapeDtypeStruct((16, 256), jnp.float32)],
      scratch_shapes=[pltpu.VMEM((16, 256), jnp.float32),
                      pltpu.VMEM((16, 256), jnp.float32)],
  )(*S_chunks, *q_chunks, deg, b, g, be, batch.reshape(GRID, 1, BLK))


@jax.jit
def _run(x, edge_index, batch, W0, b0, g0, be0, W1, b1, g1, be1,
         W2, b2, g2, be2, W3, b3, g3, be3):
  src = edge_index[0].astype(jnp.int32)
  dst = edge_index[1].astype(jnp.int32)
  pad = EP - E
  src3 = jnp.concatenate([src, jnp.zeros((pad,), jnp.int32)]).reshape(
      16, NB_TILE, BB)
  dst3 = jnp.concatenate([dst, jnp.full((pad,), N, jnp.int32)]).reshape(
      16, NB_TILE, BB)
  zeros128 = jnp.zeros((BB, 128), jnp.float32)
  ones128 = jnp.ones((BB, 128), jnp.float32)

  deg_a, deg_b = _sc_degree(dst3, ones128, zeros128)

  r2 = lambda a: a.reshape(1, -1)
  dinv_arr, h0c0, h0c1 = _tc_pre(x, deg_a, deg_b)
  h0 = [h0c0, h0c1]
  deg = dinv_arr
  S0 = _sc_propagate(h0, src3, dst3, zeros128)
  h1 = _tc_layer(S0, h0, deg, W0, r2(b0), r2(g0), r2(be0))
  S1 = _sc_propagate(h1, src3, dst3, zeros128)
  h2 = _tc_layer(S1, h1, deg, W1, r2(b1), r2(g1), r2(be1))
  S2 = _sc_propagate(h2, src3, dst3, zeros128)
  q = _tc_layer(S2, h2, deg, W2, r2(b2), r2(g2), r2(be2), W_extra=W3)
  S3 = _sc_propagate(q, src3, dst3, zeros128)
  node_emb, graph_emb = _tc_final(S3, q, deg, r2(b3), r2(g3), r2(be3), batch)
  return node_emb, graph_emb


def kernel(x, edge_index, batch, W0, b0, g0, be0, W1, b1, g1, be1,
           W2, b2, g2, be2, W3, b3, g3, be3):
  return _run(x, edge_index, batch, W0, b0, g0, be0, W1, b1, g1, be1,
              W2, b2, g2, be2, W3, b3, g3, be3)


# speedup vs baseline: 1.0412x; 1.0412x over previous
import functools

import jax
import jax.numpy as jnp
from jax import lax
from jax.experimental import pallas as pl
from jax.experimental.pallas import tpu as pltpu
from jax.experimental.pallas import tpu_sc as plsc

N = 10000
E = 160000
NB_TILE = 80
BB = 128
EP = 16 * NB_TILE * BB
NACC = 10112
HB = NB_TILE // 2
ZSPLIT = (128, 128, 128, 128, 120)

@functools.lru_cache(maxsize=None)
def _mesh():
  return plsc.VectorSubcoreMesh(
      core_axis_name="c", subcore_axis_name="s", num_cores=2, num_subcores=16)

BLK = 2000
GRID = N // BLK


def _zero_acc(acc, zb, s):
  off = 0
  for rows in ZSPLIT:
    pltpu.sync_copy(zb.at[pl.ds(0, rows)], acc.at[pl.ds(s * 632 + off, rows)])
    off += rows


def _sc_degree_body(dst_hbm, ones_hbm, z_hbm, deg_out, acc, idxb, onesb, zb):
  c = lax.axis_index("c")
  s = lax.axis_index("s")
  pltpu.sync_copy(dst_hbm.at[s], idxb)
  pltpu.sync_copy(ones_hbm, onesb)
  pltpu.sync_copy(z_hbm, zb)
  _zero_acc(acc, zb, s)
  plsc.subcore_barrier()

  @pl.when(c == 0)
  def _():
    @pl.loop(0, NB_TILE)
    def _(b):
      pltpu.sync_copy(onesb, acc.at[idxb.at[b]], add=True)

  plsc.subcore_barrier()

  @pl.when(c == 0)
  def _():
    pltpu.sync_copy(acc.at[pl.ds(s * 632, 632)], deg_out.at[pl.ds(s * 632, 632)])


def _sc_degree(dst3, ones128, zeros128):
  f = pl.kernel(
      _sc_degree_body,
      out_type=jax.ShapeDtypeStruct((NACC, 128), jnp.float32),
      mesh=_mesh(),
      scratch_types=[
          pltpu.VMEM_SHARED((NACC, 128), jnp.float32),
          pltpu.VMEM((NB_TILE, BB), jnp.int32),
          pltpu.VMEM((BB, 128), jnp.float32),
          pltpu.VMEM((BB, 128), jnp.float32),
      ],
  )
  return f(dst3, ones128, zeros128)


def _scatter_chunk(href, acc, src_hbm, dst_hbm, s, srcb, dstb, rowa, rowbuf2,
                   sema, semb, semsa, semsb):
  for half in range(2):
    pltpu.sync_copy(src_hbm.at[s, pl.ds(half * HB, HB)], srcb)
    pltpu.sync_copy(dst_hbm.at[s, pl.ds(half * HB, HB)], dstb)
    pltpu.async_copy(href.at[srcb.at[0]], rowa, sema)

    @pl.loop(0, HB, step=2, unroll=2)
    def _(b):
      pltpu.make_async_copy(href.at[srcb.at[b]], rowa, sema).wait()

      @pl.when(b > 0)
      def _():
        pltpu.make_async_copy(rowbuf2, acc.at[dstb.at[0]], semsb).wait()

      pltpu.async_copy(href.at[srcb.at[b + 1]], rowbuf2, semb)
      pltpu.async_copy(rowa, acc.at[dstb.at[b]], semsa, add=True)
      pltpu.make_async_copy(href.at[srcb.at[b + 1]], rowbuf2, semb).wait()
      pltpu.make_async_copy(rowa, acc.at[dstb.at[0]], semsa).wait()

      @pl.when(b + 2 < HB)
      def _():
        pltpu.async_copy(href.at[srcb.at[b + 2]], rowa, sema)

      pltpu.async_copy(rowbuf2, acc.at[dstb.at[b + 1]], semsb, add=True)

    pltpu.make_async_copy(rowbuf2, acc.at[dstb.at[0]], semsb).wait()


def _make_propagate_body(K):
  R = K // 2

  def body(*refs):
    hc = refs[:K]
    src_hbm, dst_hbm, z_hbm = refs[K:K + 3]
    outs = refs[K + 3:K + 3 + K]
    acc, srcb, dstb, rowa, rowbuf2, sema, semb, semsa, semsb = refs[K + 3 + K:]
    c = lax.axis_index("c")
    s = lax.axis_index("s")
    pltpu.sync_copy(z_hbm, rowa)
    _zero_acc(acc, rowa, s)
    plsc.subcore_barrier()
    for r in range(R):
      for k in range(K):
        @pl.when(c * R + r == k)
        def _(k=k):
          _scatter_chunk(hc[k], acc, src_hbm, dst_hbm, s, srcb, dstb, rowa,
                         rowbuf2, sema, semb, semsa, semsb)
      plsc.subcore_barrier()
      for k in range(K):
        @pl.when(c * R + r == k)
        def _(k=k):
          pltpu.sync_copy(acc.at[pl.ds(s * 632, 632)],
                          outs[k].at[pl.ds(s * 632, 632)])
      if r + 1 < R:
        pltpu.sync_copy(z_hbm, rowa)
        _zero_acc(acc, rowa, s)
      plsc.subcore_barrier()

  return body


def _sc_propagate(h_chunks, src3, dst3, zeros128):
  K = len(h_chunks)
  f = pl.kernel(
      _make_propagate_body(K),
      out_type=[jax.ShapeDtypeStruct((NACC, 128), jnp.float32)] * K,
      mesh=_mesh(),
      scratch_types=[
          pltpu.VMEM_SHARED((NACC, 128), jnp.float32),
          pltpu.VMEM((HB, BB), jnp.int32),
          pltpu.VMEM((HB, BB), jnp.int32),
          pltpu.VMEM((BB, 128), jnp.float32),
          pltpu.VMEM((BB, 128), jnp.float32),
          pltpu.SemaphoreType.DMA,
          pltpu.SemaphoreType.DMA,
          pltpu.SemaphoreType.DMA,
          pltpu.SemaphoreType.DMA,
      ],
  )
  return f(*h_chunks, src3, dst3, zeros128)


def _tc_pre(x, deg):
  def body(x_ref, deg_ref, o0, o1):
    dinv = lax.rsqrt(deg_ref[:, :1] + 1.0)
    h = x_ref[...] * dinv
    o0[...] = h[:, :128]
    o1[...] = h[:, 128:]

  return pl.pallas_call(
      body,
      grid=(GRID,),
      in_specs=[pl.BlockSpec((BLK, 256), lambda i: (i, 0)),
                pl.BlockSpec((BLK, 128), lambda i: (i, 0))],
      out_specs=[pl.BlockSpec((BLK, 128), lambda i: (i, 0))] * 2,
      out_shape=[jax.ShapeDtypeStruct((N, 128), jnp.float32)] * 2,
  )(x, deg)


def _tc_layer(S_chunks, h_chunks, deg, W, b, g, be, W_extra=None):
  KI = len(S_chunks)
  Wout = W.shape[1]
  KO = (W_extra.shape[1] if W_extra is not None else Wout) // 128
  n_extra = 1 if W_extra is not None else 0

  def body(*refs):
    S = refs[:KI]
    h = refs[KI:2 * KI]
    deg_ref, W_ref, b_ref, g_ref, be_ref = refs[2 * KI:2 * KI + 5]
    We_ref = refs[2 * KI + 5] if n_extra else None
    outs = refs[2 * KI + 5 + n_extra:]
    dinv = lax.rsqrt(deg_ref[:, :1] + 1.0)
    u = jnp.concatenate([S[i][...] + h[i][...] for i in range(KI)],
                        axis=1) * dinv
    z = jnp.dot(u, W_ref[...], preferred_element_type=jnp.float32) + b_ref[...]
    m = jnp.mean(z, axis=1, keepdims=True)
    d = z - m
    v = jnp.mean(d * d, axis=1, keepdims=True)
    a = d * lax.rsqrt(v + 1e-5) * g_ref[...] + be_ref[...]
    hn = jnp.maximum(a, 0.0) * dinv
    if n_extra:
      hn = jnp.dot(hn, We_ref[...], preferred_element_type=jnp.float32)
    for j in range(KO):
      outs[j][...] = hn[:, j * 128:(j + 1) * 128]

  Win = KI * 128
  in_specs = (
      [pl.BlockSpec((BLK, 128), lambda i: (i, 0))] * (2 * KI)
      + [pl.BlockSpec((BLK, 128), lambda i: (i, 0)),
         pl.BlockSpec((Win, Wout), lambda i: (0, 0)),
         pl.BlockSpec((1, Wout), lambda i: (0, 0)),
         pl.BlockSpec((1, Wout), lambda i: (0, 0)),
         pl.BlockSpec((1, Wout), lambda i: (0, 0))])
  args = list(S_chunks) + list(h_chunks) + [deg, W, b, g, be]
  if n_extra:
    in_specs.append(pl.BlockSpec(W_extra.shape, lambda i: (0, 0)))
    args.append(W_extra)
  return pl.pallas_call(
      body,
      grid=(GRID,),
      in_specs=in_specs,
      out_specs=[pl.BlockSpec((BLK, 128), lambda i: (i, 0))] * KO,
      out_shape=[jax.ShapeDtypeStruct((N, 128), jnp.float32)] * KO,
  )(*args)


def _tc_final(S_chunks, q_chunks, deg, b, g, be, batch):
  def body(S0, S1, q0, q1, deg_ref, b_ref, g_ref, be_ref, batch_ref,
           emb_out, ge_out, sums, cnts):
    i = pl.program_id(0)
    dinv = lax.rsqrt(deg_ref[:, :1] + 1.0)
    u = jnp.concatenate(
        [S0[...] + q0[...], S1[...] + q1[...]], axis=1) * dinv + b_ref[...]
    m = jnp.mean(u, axis=1, keepdims=True)
    d = u - m
    v = jnp.mean(d * d, axis=1, keepdims=True)
    hn = d * lax.rsqrt(v + 1e-5) * g_ref[...] + be_ref[...]
    hn = jnp.where(hn > 0, hn, 0.2 * hn)
    emb_out[...] = hn

    bvec = batch_ref[...].reshape(BLK)
    oh = (lax.broadcasted_iota(jnp.int32, (16, BLK), 0)
          == bvec[None, :]).astype(jnp.float32)
    ps = jnp.dot(oh, hn, preferred_element_type=jnp.float32)
    pc = jnp.sum(oh, axis=1, keepdims=True)

    @pl.when(i == 0)
    def _():
      sums[...] = jnp.zeros_like(sums)
      cnts[...] = jnp.zeros_like(cnts)

    sums[...] += ps
    cnts[...] = cnts[...] + pc
    ge_out[...] = sums[...] / jnp.maximum(cnts[...], 1.0)

  return pl.pallas_call(
      body,
      grid=(GRID,),
      in_specs=([pl.BlockSpec((BLK, 128), lambda i: (i, 0))] * 4
                + [pl.BlockSpec((BLK, 128), lambda i: (i, 0)),
                   pl.BlockSpec((1, 256), lambda i: (0, 0)),
                   pl.BlockSpec((1, 256), lambda i: (0, 0)),
                   pl.BlockSpec((1, 256), lambda i: (0, 0)),
                   pl.BlockSpec((1, 1, BLK), lambda i: (i, 0, 0))]),
      out_specs=[pl.BlockSpec((BLK, 256), lambda i: (i, 0)),
                 pl.BlockSpec((16, 256), lambda i: (0, 0))],
      out_shape=[jax.ShapeDtypeStruct((N, 256), jnp.float32),
                 jax.ShapeDtypeStruct((16, 256), jnp.float32)],
      scratch_shapes=[pltpu.VMEM((16, 256), jnp.float32),
                      pltpu.VMEM((16, 256), jnp.float32)],
  )(*S_chunks, *q_chunks, deg, b, g, be, batch.reshape(GRID, 1, BLK))


@jax.jit
def _run(x, edge_index, batch, W0, b0, g0, be0, W1, b1, g1, be1,
         W2, b2, g2, be2, W3, b3, g3, be3):
  src = edge_index[0].astype(jnp.int32)
  dst = edge_index[1].astype(jnp.int32)
  pad = EP - E
  src3 = jnp.concatenate([src, jnp.zeros((pad,), jnp.int32)]).reshape(
      16, NB_TILE, BB)
  dst3 = jnp.concatenate([dst, jnp.full((pad,), N, jnp.int32)]).reshape(
      16, NB_TILE, BB)
  zeros128 = jnp.zeros((BB, 128), jnp.float32)
  ones128 = jnp.ones((BB, 128), jnp.float32)

  deg = _sc_degree(dst3, ones128, zeros128)

  r2 = lambda a: a.reshape(1, -1)
  h0 = _tc_pre(x, deg)
  S0 = _sc_propagate(h0, src3, dst3, zeros128)
  h1 = _tc_layer(S0, h0, deg, W0, r2(b0), r2(g0), r2(be0))
  S1 = _sc_propagate(h1, src3, dst3, zeros128)
  h2 = _tc_layer(S1, h1, deg, W1, r2(b1), r2(g1), r2(be1))
  S2 = _sc_propagate(h2, src3, dst3, zeros128)
  q = _tc_layer(S2, h2, deg, W2, r2(b2), r2(g2), r2(be2), W_extra=W3)
  S3 = _sc_propagate(q, src3, dst3, zeros128)
  node_emb, graph_emb = _tc_final(S3, q, deg, r2(b3), r2(g3), r2(be3), batch)
  return node_emb, graph_emb


def kernel(x, edge_index, batch, W0, b0, g0, be0, W1, b1, g1, be1,
           W2, b2, g2, be2, W3, b3, g3, be3):
  return _run(x, edge_index, batch, W0, b0, g0, be0, W1, b1, g1, be1,
              W2, b2, g2, be2, W3, b3, g3, be3)
